# baseline (speedup 1.0000x reference)
import jax
import jax.numpy as jnp
from jax import lax
from jax.experimental import pallas as pl
from jax.experimental.pallas import tpu as pltpu
from jax.experimental.pallas import tpu_sc as plsc

EMB = 1024
NC = 2
NS = 16
NW = NC * NS

B_TOTAL = 4 * 8192
B_PER_W = B_TOTAL // NW
CHUNK = 16
N_CHUNKS = B_PER_W // CHUNK
NBUF = 7


def _gather_body(idx_hbm, table_hbm, out_hbm, idx_v, *scratch):
    bufs = scratch[:NBUF]
    insems = scratch[NBUF:2 * NBUF]
    outsems = scratch[2 * NBUF:]
    wid = lax.axis_index("s") * NC + lax.axis_index("c")
    base = wid * B_PER_W

    pltpu.sync_copy(idx_hbm.at[pl.ds(wid * N_CHUNKS, N_CHUNKS)], idx_v)

    def start_in(b, g):
        pltpu.async_copy(table_hbm.at[idx_v.at[g]], bufs[b], insems[b])

    def wait_in(b):
        pltpu.make_async_copy(table_hbm.at[idx_v.at[0]], bufs[b],
                              insems[b]).wait()

    def start_out(b, g):
        pltpu.async_copy(bufs[b], out_hbm.at[pl.ds(base + g * CHUNK, CHUNK)],
                         outsems[b])

    def wait_out(b):
        pltpu.make_async_copy(out_hbm.at[pl.ds(base, CHUNK)], bufs[b],
                              outsems[b]).wait()

    def emit(g, b, first=False, startin=True):
        wait_in(b)
        if not first:
            wait_out((b + NBUF - 1) % NBUF)
        start_out(b, g)
        if startin:
            start_in((b + NBUF - 1) % NBUF, g + NBUF - 1)

    for b in range(NBUF - 1):
        start_in(b, b)
    emit(0, 0, first=True)

    n_groups = (N_CHUNKS - NBUF) // NBUF
    g_lo = 1

    def group_step(p, carry):
        g0 = g_lo + NBUF * p
        for j in range(NBUF):
            emit(g0 + j, (g_lo + j) % NBUF)
        return carry

    if n_groups > 0:
        lax.fori_loop(0, n_groups, group_step, 0)
    g_tail = g_lo + NBUF * n_groups
    for g in range(g_tail, N_CHUNKS):
        emit(g, g % NBUF, startin=(g + NBUF - 1 < N_CHUNKS))
    wait_out((N_CHUNKS - 1) % NBUF)


@jax.jit
def _gather_call(idx2d, table):
    mesh = plsc.VectorSubcoreMesh(
        core_axis_name="c", subcore_axis_name="s",
        num_cores=NC, num_subcores=NS)
    return pl.kernel(
        _gather_body,
        out_type=jax.ShapeDtypeStruct((B_TOTAL, EMB), jnp.float32),
        mesh=mesh,
        scratch_types=(
            [pltpu.VMEM((N_CHUNKS, CHUNK), jnp.int32)]
            + [pltpu.VMEM((CHUNK, EMB), jnp.float32) for _ in range(NBUF)]
            + [pltpu.SemaphoreType.DMA for _ in range(2 * NBUF)]
        ),
    )(idx2d, table)


def kernel(position_ids, embeddings_table):
    batch, seq = position_ids.shape
    idx2d = position_ids.reshape(B_TOTAL // CHUNK, CHUNK)
    out = _gather_call(idx2d, embeddings_table)
    return out.reshape(batch, seq, EMB)
